# baseline (speedup 1.0000x reference)
import functools

import jax
import jax.numpy as jnp
from jax import lax
from jax.experimental import pallas as pl
from jax.experimental.pallas import tpu as pltpu
from jax.experimental.pallas import tpu_sc as plsc

V = 100000
D = 512
H = 4
L = 4
CHUNK = 256
B = 32

NC = 2
NS = 16
NW = NC * NS

N_IDS = B * CHUNK
ROWS_PER_W = N_IDS // NW
GCHUNK = 64
NCH = ROWS_PER_W // GCHUNK

BV = 4000
NBLK = V // BV

L0_CHUNKS = 4


def _sc_gather_body(emb_hbm, idx_hbm, out_hbm, idx_v, rows_v, sem):
    wid = lax.axis_index("s") * NC + lax.axis_index("c")
    pltpu.sync_copy(idx_hbm.at[pl.ds(wid * NCH, NCH)], idx_v)
    for c in range(NCH):
        cp = pltpu.async_copy(emb_hbm.at[idx_v.at[c]], rows_v.at[c % 2], sem)
        cp.wait()
        base = wid * ROWS_PER_W + c * GCHUNK
        pltpu.sync_copy(rows_v.at[c % 2], out_hbm.at[pl.ds(base, GCHUNK)])


@functools.cache
def _sc_gather():
    return pl.kernel(
        _sc_gather_body,
        mesh=plsc.VectorSubcoreMesh(core_axis_name="c", subcore_axis_name="s"),
        out_type=jax.ShapeDtypeStruct((N_IDS, D), jnp.float32),
        scratch_types=[
            pltpu.VMEM((NCH, GCHUNK), jnp.int32),
            pltpu.VMEM((2, GCHUNK, D), jnp.float32),
            pltpu.SemaphoreType.DMA,
        ],
    )


def _sc_gather_small_body(emb_hbm, idx_hbm, out_hbm, idx_v, rows_v, sem):
    wid = lax.axis_index("s") * NC + lax.axis_index("c")

    @pl.when(wid == 0)
    def _():
        pltpu.sync_copy(idx_hbm, idx_v)
        pltpu.async_copy(emb_hbm.at[idx_v], rows_v, sem).wait()
        pltpu.sync_copy(rows_v, out_hbm)


@functools.cache
def _sc_gather_small():
    return pl.kernel(
        _sc_gather_small_body,
        mesh=plsc.VectorSubcoreMesh(core_axis_name="c", subcore_axis_name="s"),
        out_type=jax.ShapeDtypeStruct((B, D), jnp.float32),
        scratch_types=[
            pltpu.VMEM((B,), jnp.int32),
            pltpu.VMEM((B, D), jnp.float32),
            pltpu.SemaphoreType.DMA,
        ],
    )


def _dot(a, b):
    return lax.dot_general(a, b, (((1,), (0,)), ((), ())),
                           preferred_element_type=jnp.float32,
                           precision=lax.Precision.HIGHEST)


def _dot_t(a, b):
    return lax.dot_general(a, b, (((1,), (1,)), ((), ())),
                           preferred_element_type=jnp.float32,
                           precision=lax.Precision.HIGHEST)


_INV_SQRT_H = 1.0 / (H ** 0.5)


def _pyr_body(x_ref, proj_ref, mem_ref, retr_out, xs_ref, enc_ref):
    l = pl.program_id(0)
    P = proj_ref[0]
    M = mem_ref[0]

    @pl.when(l == 0)
    def _layer0():
        enc_ref[...] = jnp.zeros((B, D), jnp.float32)
        cur = CHUNK // H
        for c in range(L0_CHUNKS):
            ng = (B * cur) // L0_CHUNKS
            xg = x_ref[pl.ds(c * ng * H, ng * H), :].reshape(ng, H * D)
            k = _dot(xg, P) * _INV_SQRT_H
            rows = lax.broadcasted_iota(jnp.int32, (B, ng), 0)
            cols = lax.broadcasted_iota(jnp.int32, (B, ng), 1)
            b0 = c * (ng // cur)
            sel = ((cols == (rows - b0) * cur + (cur - 1))
                   & (rows >= b0) & (rows < b0 + ng // cur))
            enc_ref[...] += _dot(sel.astype(jnp.float32), k)
            xs_ref[pl.ds(c * ng, ng), :] = _dot(k, M)

    def _mid_layer(n_in):
        n = n_in // H
        cur = n // B
        xg = xs_ref[pl.ds(0, n_in), :].reshape(n, H * D)
        k = _dot(xg, P) * _INV_SQRT_H
        rows = lax.broadcasted_iota(jnp.int32, (B, n), 0)
        cols = lax.broadcasted_iota(jnp.int32, (B, n), 1)
        sel = (cols == rows * cur + (cur - 1)).astype(jnp.float32)
        enc_ref[...] += _dot(sel, k)
        xs_ref[pl.ds(0, n), :] = _dot(k, M)

    @pl.when(l == 1)
    def _layer1():
        _mid_layer(2048)

    @pl.when(l == 2)
    def _layer2():
        _mid_layer(512)

    @pl.when(l == 3)
    def _layer3():
        xg = xs_ref[pl.ds(0, 128), :].reshape(B, H * D)
        k = _dot(xg, P) * _INV_SQRT_H
        enc = enc_ref[...] + k
        retr_out[...] = _dot(enc, M)


def _tc_pyramid(x, proj2, mem):
    return pl.pallas_call(
        _pyr_body,
        grid=(L,),
        in_specs=[
            pl.BlockSpec((N_IDS, D), lambda l: (0, 0)),
            pl.BlockSpec((1, H * D, D), lambda l: (l, 0, 0)),
            pl.BlockSpec((1, D, D), lambda l: (l, 0, 0)),
        ],
        out_specs=pl.BlockSpec((B, D), lambda l: (0, 0)),
        out_shape=jax.ShapeDtypeStruct((B, D), jnp.float32),
        scratch_shapes=[
            pltpu.VMEM((B * CHUNK // H, D), jnp.float32),
            pltpu.VMEM((B, D), jnp.float32),
        ],
    )(x, proj2, mem)


def _dec_body(retr_ref, emb_ref, val_out, idx_out, bval_ref, bidx_ref):
    i = pl.program_id(0)

    @pl.when(i == 0)
    def _init():
        bval_ref[...] = jnp.full((B,), -jnp.inf, jnp.float32)
        bidx_ref[...] = jnp.zeros((B,), jnp.int32)

    scores = _dot_t(retr_ref[...], emb_ref[...])
    bm = jnp.max(scores, axis=1)
    io = lax.broadcasted_iota(jnp.int32, (B, BV), 1)
    ba = jnp.min(jnp.where(scores == bm[:, None], io, V), axis=1) + i * BV
    better = bm > bval_ref[...]
    nv = jnp.where(better, bm, bval_ref[...])
    ni = jnp.where(better, ba, bidx_ref[...])
    bval_ref[...] = nv
    bidx_ref[...] = ni
    val_out[...] = nv
    idx_out[...] = ni


def _tc_decode(retr, emb):
    return pl.pallas_call(
        _dec_body,
        grid=(NBLK,),
        in_specs=[
            pl.BlockSpec((B, D), lambda i: (0, 0)),
            pl.BlockSpec((BV, D), lambda i: (i, 0)),
        ],
        out_specs=[
            pl.BlockSpec((B,), lambda i: (0,)),
            pl.BlockSpec((B,), lambda i: (0,)),
        ],
        out_shape=[
            jax.ShapeDtypeStruct((B,), jnp.float32),
            jax.ShapeDtypeStruct((B,), jnp.int32),
        ],
        scratch_shapes=[
            pltpu.VMEM((B,), jnp.float32),
            pltpu.VMEM((B,), jnp.int32),
        ],
    )(retr, emb)


def kernel(input_sequence_ids, emb, proj, mem):
    ids = input_sequence_ids.reshape(NW * NCH, GCHUNK).astype(jnp.int32)
    x = _sc_gather()(emb, ids)
    retr = _tc_pyramid(x, proj.reshape(L, H * D, D), mem)
    score_max, tok = _tc_decode(retr, emb)
    out_vec = _sc_gather_small()(emb, tok)
    return (out_vec, score_max)


# speedup vs baseline: 3.1156x; 3.1156x over previous
import functools

import jax
import jax.numpy as jnp
from jax import lax
from jax.experimental import pallas as pl
from jax.experimental.pallas import tpu as pltpu
from jax.experimental.pallas import tpu_sc as plsc

V = 100000
D = 512
H = 4
L = 4
CHUNK = 256
B = 32

NC = 2
NS = 16
NW = NC * NS

N_IDS = B * CHUNK
ROWS_PER_W = N_IDS // NW
GCHUNK = 64
NCH = ROWS_PER_W // GCHUNK

BV = 4000
NBLK = V // BV

L0_CHUNKS = 4


def _sc_gather_body(emb_hbm, idx_hbm, out_hbm, idx_v, rows_v, sem):
    wid = lax.axis_index("s") * NC + lax.axis_index("c")
    pltpu.sync_copy(idx_hbm.at[pl.ds(wid * NCH, NCH)], idx_v)
    for c in range(NCH):
        cp = pltpu.async_copy(emb_hbm.at[idx_v.at[c]], rows_v.at[c % 2], sem)
        cp.wait()
        base = wid * ROWS_PER_W + c * GCHUNK
        pltpu.sync_copy(rows_v.at[c % 2], out_hbm.at[pl.ds(base, GCHUNK)])


@functools.cache
def _sc_gather():
    return pl.kernel(
        _sc_gather_body,
        mesh=plsc.VectorSubcoreMesh(core_axis_name="c", subcore_axis_name="s"),
        out_type=jax.ShapeDtypeStruct((N_IDS, D), jnp.float32),
        scratch_types=[
            pltpu.VMEM((NCH, GCHUNK), jnp.int32),
            pltpu.VMEM((2, GCHUNK, D), jnp.float32),
            pltpu.SemaphoreType.DMA,
        ],
    )


def _sc_gather_small_body(emb_hbm, idx_hbm, out_hbm, idx_v, rows_v, sem):
    wid = lax.axis_index("s") * NC + lax.axis_index("c")

    @pl.when(wid == 0)
    def _():
        pltpu.sync_copy(idx_hbm, idx_v)
        pltpu.async_copy(emb_hbm.at[idx_v], rows_v, sem).wait()
        pltpu.sync_copy(rows_v, out_hbm)


@functools.cache
def _sc_gather_small():
    return pl.kernel(
        _sc_gather_small_body,
        mesh=plsc.VectorSubcoreMesh(core_axis_name="c", subcore_axis_name="s"),
        out_type=jax.ShapeDtypeStruct((B, D), jnp.float32),
        scratch_types=[
            pltpu.VMEM((B,), jnp.int32),
            pltpu.VMEM((B, D), jnp.float32),
            pltpu.SemaphoreType.DMA,
        ],
    )


def _dot(a, b):
    return lax.dot_general(a, b, (((1,), (0,)), ((), ())),
                           preferred_element_type=jnp.float32,
                           precision=lax.Precision.DEFAULT)


def _dot_t(a, b):
    return lax.dot_general(a, b, (((1,), (1,)), ((), ())),
                           preferred_element_type=jnp.float32,
                           precision=lax.Precision.DEFAULT)


_INV_SQRT_H = 1.0 / (H ** 0.5)


def _pyr_body(x_ref, proj_ref, mem_ref, retr_out, xs_ref, enc_ref):
    l = pl.program_id(0)
    P = proj_ref[0]
    M = mem_ref[0]

    @pl.when(l == 0)
    def _layer0():
        enc_ref[...] = jnp.zeros((B, D), jnp.float32)
        cur = CHUNK // H
        for c in range(L0_CHUNKS):
            ng = (B * cur) // L0_CHUNKS
            xg = x_ref[pl.ds(c * ng * H, ng * H), :].reshape(ng, H * D)
            k = _dot(xg, P) * _INV_SQRT_H
            rows = lax.broadcasted_iota(jnp.int32, (B, ng), 0)
            cols = lax.broadcasted_iota(jnp.int32, (B, ng), 1)
            b0 = c * (ng // cur)
            sel = ((cols == (rows - b0) * cur + (cur - 1))
                   & (rows >= b0) & (rows < b0 + ng // cur))
            enc_ref[...] += _dot(sel.astype(jnp.float32), k)
            xs_ref[pl.ds(c * ng, ng), :] = _dot(k, M)

    def _mid_layer(n_in):
        n = n_in // H
        cur = n // B
        xg = xs_ref[pl.ds(0, n_in), :].reshape(n, H * D)
        k = _dot(xg, P) * _INV_SQRT_H
        rows = lax.broadcasted_iota(jnp.int32, (B, n), 0)
        cols = lax.broadcasted_iota(jnp.int32, (B, n), 1)
        sel = (cols == rows * cur + (cur - 1)).astype(jnp.float32)
        enc_ref[...] += _dot(sel, k)
        xs_ref[pl.ds(0, n), :] = _dot(k, M)

    @pl.when(l == 1)
    def _layer1():
        _mid_layer(2048)

    @pl.when(l == 2)
    def _layer2():
        _mid_layer(512)

    @pl.when(l == 3)
    def _layer3():
        xg = xs_ref[pl.ds(0, 128), :].reshape(B, H * D)
        k = _dot(xg, P) * _INV_SQRT_H
        enc = enc_ref[...] + k
        retr_out[...] = _dot(enc, M)


def _tc_pyramid(x, proj2, mem):
    return pl.pallas_call(
        _pyr_body,
        grid=(L,),
        in_specs=[
            pl.BlockSpec((N_IDS, D), lambda l: (0, 0)),
            pl.BlockSpec((1, H * D, D), lambda l: (l, 0, 0)),
            pl.BlockSpec((1, D, D), lambda l: (l, 0, 0)),
        ],
        out_specs=pl.BlockSpec((B, D), lambda l: (0, 0)),
        out_shape=jax.ShapeDtypeStruct((B, D), jnp.float32),
        scratch_shapes=[
            pltpu.VMEM((B * CHUNK // H, D), jnp.float32),
            pltpu.VMEM((B, D), jnp.float32),
        ],
    )(x, proj2, mem)


def _dec_body(retr_ref, emb_ref, val_out, idx_out, bval_ref, bidx_ref):
    i = pl.program_id(0)

    @pl.when(i == 0)
    def _init():
        bval_ref[...] = jnp.full((B,), -jnp.inf, jnp.float32)
        bidx_ref[...] = jnp.zeros((B,), jnp.int32)

    scores = _dot_t(retr_ref[...], emb_ref[...])
    bm = jnp.max(scores, axis=1)
    io = lax.broadcasted_iota(jnp.int32, (B, BV), 1)
    ba = jnp.min(jnp.where(scores == bm[:, None], io, V), axis=1) + i * BV
    better = bm > bval_ref[...]
    nv = jnp.where(better, bm, bval_ref[...])
    ni = jnp.where(better, ba, bidx_ref[...])
    bval_ref[...] = nv
    bidx_ref[...] = ni
    val_out[...] = nv
    idx_out[...] = ni


def _tc_decode(retr, emb):
    return pl.pallas_call(
        _dec_body,
        grid=(NBLK,),
        in_specs=[
            pl.BlockSpec((B, D), lambda i: (0, 0)),
            pl.BlockSpec((BV, D), lambda i: (i, 0)),
        ],
        out_specs=[
            pl.BlockSpec((B,), lambda i: (0,)),
            pl.BlockSpec((B,), lambda i: (0,)),
        ],
        out_shape=[
            jax.ShapeDtypeStruct((B,), jnp.float32),
            jax.ShapeDtypeStruct((B,), jnp.int32),
        ],
        scratch_shapes=[
            pltpu.VMEM((B,), jnp.float32),
            pltpu.VMEM((B,), jnp.int32),
        ],
    )(retr, emb)


def kernel(input_sequence_ids, emb, proj, mem):
    ids = input_sequence_ids.reshape(NW * NCH, GCHUNK).astype(jnp.int32)
    x = _sc_gather()(emb, ids)
    retr = _tc_pyramid(x, proj.reshape(L, H * D, D), mem)
    score_max, tok = _tc_decode(retr, emb)
    out_vec = _sc_gather_small()(emb, tok)
    return (out_vec, score_max)


# speedup vs baseline: 3.1340x; 1.0059x over previous
import functools

import jax
import jax.numpy as jnp
from jax import lax
from jax.experimental import pallas as pl
from jax.experimental.pallas import tpu as pltpu
from jax.experimental.pallas import tpu_sc as plsc

V = 100000
D = 512
H = 4
L = 4
CHUNK = 256
B = 32

NC = 2
NS = 16
NW = NC * NS

N_IDS = B * CHUNK
ROWS_PER_W = N_IDS // NW
GCHUNK = 64
NCH = ROWS_PER_W // GCHUNK

BV = 10000
NBLK = V // BV

L0_CHUNKS = 4


def _sc_gather_body(emb_hbm, idx_hbm, out_hbm, idx_v, rows_v, sem):
    wid = lax.axis_index("s") * NC + lax.axis_index("c")
    pltpu.sync_copy(idx_hbm.at[pl.ds(wid * NCH, NCH)], idx_v)
    for c in range(NCH):
        cp = pltpu.async_copy(emb_hbm.at[idx_v.at[c]], rows_v.at[c % 2], sem)
        cp.wait()
        base = wid * ROWS_PER_W + c * GCHUNK
        pltpu.sync_copy(rows_v.at[c % 2], out_hbm.at[pl.ds(base, GCHUNK)])


@functools.cache
def _sc_gather():
    return pl.kernel(
        _sc_gather_body,
        mesh=plsc.VectorSubcoreMesh(core_axis_name="c", subcore_axis_name="s"),
        out_type=jax.ShapeDtypeStruct((N_IDS, D), jnp.float32),
        scratch_types=[
            pltpu.VMEM((NCH, GCHUNK), jnp.int32),
            pltpu.VMEM((2, GCHUNK, D), jnp.float32),
            pltpu.SemaphoreType.DMA,
        ],
    )


def _sc_gather_small_body(emb_hbm, idx_hbm, out_hbm, idx_v, rows_v, sem):
    wid = lax.axis_index("s") * NC + lax.axis_index("c")

    @pl.when(wid == 0)
    def _():
        pltpu.sync_copy(idx_hbm, idx_v)
        pltpu.async_copy(emb_hbm.at[idx_v], rows_v, sem).wait()
        pltpu.sync_copy(rows_v, out_hbm)


@functools.cache
def _sc_gather_small():
    return pl.kernel(
        _sc_gather_small_body,
        mesh=plsc.VectorSubcoreMesh(core_axis_name="c", subcore_axis_name="s"),
        out_type=jax.ShapeDtypeStruct((B, D), jnp.float32),
        scratch_types=[
            pltpu.VMEM((B,), jnp.int32),
            pltpu.VMEM((B, D), jnp.float32),
            pltpu.SemaphoreType.DMA,
        ],
    )


def _dot(a, b):
    return lax.dot_general(a, b, (((1,), (0,)), ((), ())),
                           preferred_element_type=jnp.float32,
                           precision=lax.Precision.DEFAULT)


def _dot_t(a, b):
    return lax.dot_general(a, b, (((1,), (1,)), ((), ())),
                           preferred_element_type=jnp.float32,
                           precision=lax.Precision.DEFAULT)


_INV_SQRT_H = 1.0 / (H ** 0.5)


def _pyr_body(x_ref, proj_ref, mem_ref, retr_out, xs_ref, enc_ref):
    l = pl.program_id(0)
    P = proj_ref[0]
    M = mem_ref[0]

    @pl.when(l == 0)
    def _layer0():
        enc_ref[...] = jnp.zeros((B, D), jnp.float32)
        cur = CHUNK // H
        for c in range(L0_CHUNKS):
            ng = (B * cur) // L0_CHUNKS
            xg = x_ref[pl.ds(c * ng * H, ng * H), :].reshape(ng, H * D)
            k = _dot(xg, P) * _INV_SQRT_H
            rows = lax.broadcasted_iota(jnp.int32, (B, ng), 0)
            cols = lax.broadcasted_iota(jnp.int32, (B, ng), 1)
            b0 = c * (ng // cur)
            sel = ((cols == (rows - b0) * cur + (cur - 1))
                   & (rows >= b0) & (rows < b0 + ng // cur))
            enc_ref[...] += _dot(sel.astype(jnp.float32), k)
            xs_ref[pl.ds(c * ng, ng), :] = _dot(k, M)

    def _mid_layer(n_in):
        n = n_in // H
        cur = n // B
        xg = xs_ref[pl.ds(0, n_in), :].reshape(n, H * D)
        k = _dot(xg, P) * _INV_SQRT_H
        rows = lax.broadcasted_iota(jnp.int32, (B, n), 0)
        cols = lax.broadcasted_iota(jnp.int32, (B, n), 1)
        sel = (cols == rows * cur + (cur - 1)).astype(jnp.float32)
        enc_ref[...] += _dot(sel, k)
        xs_ref[pl.ds(0, n), :] = _dot(k, M)

    @pl.when(l == 1)
    def _layer1():
        _mid_layer(2048)

    @pl.when(l == 2)
    def _layer2():
        _mid_layer(512)

    @pl.when(l == 3)
    def _layer3():
        xg = xs_ref[pl.ds(0, 128), :].reshape(B, H * D)
        k = _dot(xg, P) * _INV_SQRT_H
        enc = enc_ref[...] + k
        retr_out[...] = _dot(enc, M)


def _tc_pyramid(x, proj2, mem):
    return pl.pallas_call(
        _pyr_body,
        grid=(L,),
        in_specs=[
            pl.BlockSpec((N_IDS, D), lambda l: (0, 0)),
            pl.BlockSpec((1, H * D, D), lambda l: (l, 0, 0)),
            pl.BlockSpec((1, D, D), lambda l: (l, 0, 0)),
        ],
        out_specs=pl.BlockSpec((B, D), lambda l: (0, 0)),
        out_shape=jax.ShapeDtypeStruct((B, D), jnp.float32),
        scratch_shapes=[
            pltpu.VMEM((B * CHUNK // H, D), jnp.float32),
            pltpu.VMEM((B, D), jnp.float32),
        ],
    )(x, proj2, mem)


def _dec_body(retr_ref, emb_ref, val_out, idx_out, bval_ref, bidx_ref):
    i = pl.program_id(0)

    @pl.when(i == 0)
    def _init():
        bval_ref[...] = jnp.full((B,), -jnp.inf, jnp.float32)
        bidx_ref[...] = jnp.zeros((B,), jnp.int32)

    scores = _dot_t(emb_ref[...], retr_ref[...])
    bm = jnp.max(scores, axis=0)
    io = lax.broadcasted_iota(jnp.int32, (BV, B), 0)
    ba = jnp.min(jnp.where(scores == bm[None, :], io, V), axis=0) + i * BV
    better = bm > bval_ref[...]
    nv = jnp.where(better, bm, bval_ref[...])
    ni = jnp.where(better, ba, bidx_ref[...])
    bval_ref[...] = nv
    bidx_ref[...] = ni
    val_out[...] = nv
    idx_out[...] = ni


def _tc_decode(retr, emb):
    return pl.pallas_call(
        _dec_body,
        grid=(NBLK,),
        in_specs=[
            pl.BlockSpec((B, D), lambda i: (0, 0)),
            pl.BlockSpec((BV, D), lambda i: (i, 0)),
        ],
        out_specs=[
            pl.BlockSpec((B,), lambda i: (0,)),
            pl.BlockSpec((B,), lambda i: (0,)),
        ],
        out_shape=[
            jax.ShapeDtypeStruct((B,), jnp.float32),
            jax.ShapeDtypeStruct((B,), jnp.int32),
        ],
        scratch_shapes=[
            pltpu.VMEM((B,), jnp.float32),
            pltpu.VMEM((B,), jnp.int32),
        ],
    )(retr, emb)


def kernel(input_sequence_ids, emb, proj, mem):
    ids = input_sequence_ids.reshape(NW * NCH, GCHUNK).astype(jnp.int32)
    x = _sc_gather()(emb, ids)
    retr = _tc_pyramid(x, proj.reshape(L, H * D, D), mem)
    score_max, tok = _tc_decode(retr, emb)
    out_vec = _sc_gather_small()(emb, tok)
    return (out_vec, score_max)


# speedup vs baseline: 3.1929x; 1.0188x over previous
import functools

import jax
import jax.numpy as jnp
from jax import lax
from jax.experimental import pallas as pl
from jax.experimental.pallas import tpu as pltpu
from jax.experimental.pallas import tpu_sc as plsc

V = 100000
D = 512
H = 4
L = 4
CHUNK = 256
B = 32

NC = 2
NS = 16
NW = NC * NS

N_IDS = B * CHUNK
ROWS_PER_W = N_IDS // NW
GCHUNK = 64
NCH = ROWS_PER_W // GCHUNK

BV = 10000
NBLK = V // BV

L0_CHUNKS = 4


def _sc_gather_body4(emb_hbm, idx_hbm, out_hbm, idx_v, rows_v,
                     gs0, gs1, os0, os1):
    wid = lax.axis_index("s") * NC + lax.axis_index("c")
    pltpu.sync_copy(idx_hbm.at[pl.ds(wid * NCH, NCH)], idx_v)
    base = wid * ROWS_PER_W

    def _gather(c, sem):
        return pltpu.async_copy(emb_hbm.at[idx_v.at[c]], rows_v.at[c % 2], sem)

    def _flush(c, sem):
        return pltpu.async_copy(rows_v.at[c % 2],
                                out_hbm.at[pl.ds(base + c * GCHUNK, GCHUNK)],
                                sem)

    g0 = _gather(0, gs0)
    g1 = _gather(1, gs1)
    g0.wait()
    o0 = _flush(0, os0)
    g1.wait()
    o1 = _flush(1, os1)
    o0.wait()
    g2 = _gather(2, gs0)
    o1.wait()
    g3 = _gather(3, gs1)
    g2.wait()
    o2 = _flush(2, os0)
    g3.wait()
    o3 = _flush(3, os1)
    o2.wait()
    o3.wait()


@functools.cache
def _sc_gather():
    return pl.kernel(
        _sc_gather_body4,
        mesh=plsc.VectorSubcoreMesh(core_axis_name="c", subcore_axis_name="s"),
        out_type=jax.ShapeDtypeStruct((N_IDS, D), jnp.float32),
        scratch_types=[
            pltpu.VMEM((NCH, GCHUNK), jnp.int32),
            pltpu.VMEM((2, GCHUNK, D), jnp.float32),
            pltpu.SemaphoreType.DMA,
            pltpu.SemaphoreType.DMA,
            pltpu.SemaphoreType.DMA,
            pltpu.SemaphoreType.DMA,
        ],
    )


def _sc_gather_small_body(emb_hbm, idx_hbm, out_hbm, idx_v, rows_v, sem):
    wid = lax.axis_index("s") * NC + lax.axis_index("c")

    @pl.when(wid == 0)
    def _():
        pltpu.sync_copy(idx_hbm, idx_v)
        pltpu.async_copy(emb_hbm.at[idx_v], rows_v, sem).wait()
        pltpu.sync_copy(rows_v, out_hbm)


@functools.cache
def _sc_gather_small():
    return pl.kernel(
        _sc_gather_small_body,
        mesh=plsc.VectorSubcoreMesh(core_axis_name="c", subcore_axis_name="s"),
        out_type=jax.ShapeDtypeStruct((B, D), jnp.float32),
        scratch_types=[
            pltpu.VMEM((B,), jnp.int32),
            pltpu.VMEM((B, D), jnp.float32),
            pltpu.SemaphoreType.DMA,
        ],
    )


def _dot(a, b):
    return lax.dot_general(a, b, (((1,), (0,)), ((), ())),
                           preferred_element_type=jnp.float32,
                           precision=lax.Precision.DEFAULT)


def _dot_t(a, b):
    return lax.dot_general(a, b, (((1,), (1,)), ((), ())),
                           preferred_element_type=jnp.float32,
                           precision=lax.Precision.DEFAULT)


_INV_SQRT_H = 1.0 / (H ** 0.5)


def _pyr_body(x_ref, proj_ref, mem_ref, retr_out, xs_ref, enc_ref):
    l = pl.program_id(0)
    P = proj_ref[0]
    M = mem_ref[0]

    @pl.when(l == 0)
    def _layer0():
        enc_ref[...] = jnp.zeros((B, D), jnp.float32)
        cur = CHUNK // H
        for c in range(L0_CHUNKS):
            ng = (B * cur) // L0_CHUNKS
            xg = x_ref[pl.ds(c * ng * H, ng * H), :].reshape(ng, H * D)
            k = _dot(xg, P) * _INV_SQRT_H
            rows = lax.broadcasted_iota(jnp.int32, (B, ng), 0)
            cols = lax.broadcasted_iota(jnp.int32, (B, ng), 1)
            b0 = c * (ng // cur)
            sel = ((cols == (rows - b0) * cur + (cur - 1))
                   & (rows >= b0) & (rows < b0 + ng // cur))
            enc_ref[...] += _dot(sel.astype(jnp.float32), k)
            xs_ref[pl.ds(c * ng, ng), :] = _dot(k, M)

    def _mid_layer(n_in):
        n = n_in // H
        cur = n // B
        xg = xs_ref[pl.ds(0, n_in), :].reshape(n, H * D)
        k = _dot(xg, P) * _INV_SQRT_H
        rows = lax.broadcasted_iota(jnp.int32, (B, n), 0)
        cols = lax.broadcasted_iota(jnp.int32, (B, n), 1)
        sel = (cols == rows * cur + (cur - 1)).astype(jnp.float32)
        enc_ref[...] += _dot(sel, k)
        xs_ref[pl.ds(0, n), :] = _dot(k, M)

    @pl.when(l == 1)
    def _layer1():
        _mid_layer(2048)

    @pl.when(l == 2)
    def _layer2():
        _mid_layer(512)

    @pl.when(l == 3)
    def _layer3():
        xg = xs_ref[pl.ds(0, 128), :].reshape(B, H * D)
        k = _dot(xg, P) * _INV_SQRT_H
        enc = enc_ref[...] + k
        retr_out[...] = _dot(enc, M)


def _tc_pyramid(x, proj2, mem):
    return pl.pallas_call(
        _pyr_body,
        grid=(L,),
        in_specs=[
            pl.BlockSpec((N_IDS, D), lambda l: (0, 0)),
            pl.BlockSpec((1, H * D, D), lambda l: (l, 0, 0)),
            pl.BlockSpec((1, D, D), lambda l: (l, 0, 0)),
        ],
        out_specs=pl.BlockSpec((B, D), lambda l: (0, 0)),
        out_shape=jax.ShapeDtypeStruct((B, D), jnp.float32),
        scratch_shapes=[
            pltpu.VMEM((B * CHUNK // H, D), jnp.float32),
            pltpu.VMEM((B, D), jnp.float32),
        ],
    )(x, proj2, mem)


def _dec_body(retr_ref, emb_ref, val_out, idx_out, bval_ref, bidx_ref):
    i = pl.program_id(0)

    @pl.when(i == 0)
    def _init():
        bval_ref[...] = jnp.full((B,), -jnp.inf, jnp.float32)
        bidx_ref[...] = jnp.zeros((B,), jnp.int32)

    scores = _dot_t(emb_ref[...], retr_ref[...])
    bm = jnp.max(scores, axis=0)
    io = lax.broadcasted_iota(jnp.int32, (BV, B), 0)
    ba = jnp.min(jnp.where(scores == bm[None, :], io, V), axis=0) + i * BV
    better = bm > bval_ref[...]
    nv = jnp.where(better, bm, bval_ref[...])
    ni = jnp.where(better, ba, bidx_ref[...])
    bval_ref[...] = nv
    bidx_ref[...] = ni
    val_out[...] = nv
    idx_out[...] = ni


def _tc_decode(retr, emb):
    return pl.pallas_call(
        _dec_body,
        grid=(NBLK,),
        in_specs=[
            pl.BlockSpec((B, D), lambda i: (0, 0)),
            pl.BlockSpec((BV, D), lambda i: (i, 0)),
        ],
        out_specs=[
            pl.BlockSpec((B,), lambda i: (0,)),
            pl.BlockSpec((B,), lambda i: (0,)),
        ],
        out_shape=[
            jax.ShapeDtypeStruct((B,), jnp.float32),
            jax.ShapeDtypeStruct((B,), jnp.int32),
        ],
        scratch_shapes=[
            pltpu.VMEM((B,), jnp.float32),
            pltpu.VMEM((B,), jnp.int32),
        ],
    )(retr, emb)


def kernel(input_sequence_ids, emb, proj, mem):
    ids = input_sequence_ids.reshape(NW * NCH, GCHUNK).astype(jnp.int32)
    x = _sc_gather()(emb, ids)
    retr = _tc_pyramid(x, proj.reshape(L, H * D, D), mem)
    score_max, tok = _tc_decode(retr, emb)
    out_vec = _sc_gather_small()(emb, tok)
    return (out_vec, score_max)


# speedup vs baseline: 3.2034x; 1.0033x over previous
import functools

import jax
import jax.numpy as jnp
from jax import lax
from jax.experimental import pallas as pl
from jax.experimental.pallas import tpu as pltpu
from jax.experimental.pallas import tpu_sc as plsc

V = 100000
D = 512
H = 4
L = 4
CHUNK = 256
B = 32

NC = 2
NS = 16
NW = NC * NS

N_IDS = B * CHUNK
ROWS_PER_W = N_IDS // NW
GCHUNK = 64
NCH = ROWS_PER_W // GCHUNK

BV = 4000
NBLK = V // BV
ERING = 3

TOK = 1024
NXC = N_IDS // TOK
XRING = 3
NG = TOK // H
CUR0 = CHUNK // H

_INV_SQRT_H = 1.0 / (H ** 0.5)


def _sc_gather_body4(emb_hbm, idx_hbm, out_hbm, idx_v, rows_v,
                     gs0, gs1, os0, os1):
    wid = lax.axis_index("s") * NC + lax.axis_index("c")
    pltpu.sync_copy(idx_hbm.at[pl.ds(wid * NCH, NCH)], idx_v)
    base = wid * ROWS_PER_W

    def _gather(c, sem):
        return pltpu.async_copy(emb_hbm.at[idx_v.at[c]], rows_v.at[c % 2], sem)

    def _flush(c, sem):
        return pltpu.async_copy(rows_v.at[c % 2],
                                out_hbm.at[pl.ds(base + c * GCHUNK, GCHUNK)],
                                sem)

    g0 = _gather(0, gs0)
    g1 = _gather(1, gs1)
    g0.wait()
    o0 = _flush(0, os0)
    g1.wait()
    o1 = _flush(1, os1)
    o0.wait()
    g2 = _gather(2, gs0)
    o1.wait()
    g3 = _gather(3, gs1)
    g2.wait()
    o2 = _flush(2, os0)
    g3.wait()
    o3 = _flush(3, os1)
    o2.wait()
    o3.wait()


@functools.cache
def _sc_gather():
    return pl.kernel(
        _sc_gather_body4,
        mesh=plsc.VectorSubcoreMesh(core_axis_name="c", subcore_axis_name="s"),
        out_type=jax.ShapeDtypeStruct((N_IDS, D), jnp.float32),
        scratch_types=[
            pltpu.VMEM((NCH, GCHUNK), jnp.int32),
            pltpu.VMEM((2, GCHUNK, D), jnp.float32),
            pltpu.SemaphoreType.DMA,
            pltpu.SemaphoreType.DMA,
            pltpu.SemaphoreType.DMA,
            pltpu.SemaphoreType.DMA,
        ],
    )


def _sc_gather_small_body(emb_hbm, idx_hbm, out_hbm, idx_v, rows_v, sem):
    wid = lax.axis_index("s") * NC + lax.axis_index("c")

    @pl.when(wid == 0)
    def _():
        pltpu.sync_copy(idx_hbm, idx_v)
        pltpu.async_copy(emb_hbm.at[idx_v], rows_v, sem).wait()
        pltpu.sync_copy(rows_v, out_hbm)


@functools.cache
def _sc_gather_small():
    return pl.kernel(
        _sc_gather_small_body,
        mesh=plsc.VectorSubcoreMesh(core_axis_name="c", subcore_axis_name="s"),
        out_type=jax.ShapeDtypeStruct((B, D), jnp.float32),
        scratch_types=[
            pltpu.VMEM((B,), jnp.int32),
            pltpu.VMEM((B, D), jnp.float32),
            pltpu.SemaphoreType.DMA,
        ],
    )


def _dot(a, b):
    return lax.dot_general(a, b, (((1,), (0,)), ((), ())),
                           preferred_element_type=jnp.float32)


def _dot_t(a, b):
    return lax.dot_general(a, b, (((1,), (1,)), ((), ())),
                           preferred_element_type=jnp.float32)


def _fused_body(x_hbm, proj_hbm, mem_ref, emb_hbm, val_out, idx_out,
                xbuf, pbuf, ebuf, xs_ref, kbuf, enc_ref, xsem, psem, esem):
    ecp = {}

    def estart(b):
        ecp[b] = pltpu.make_async_copy(
            emb_hbm.at[pl.ds(b * BV, BV), :], ebuf.at[b % ERING],
            esem.at[b % ERING])
        ecp[b].start()

    xcp = {}

    def xstart(c):
        xcp[c] = pltpu.make_async_copy(
            x_hbm.at[pl.ds(c * TOK, TOK), :], xbuf.at[c % XRING],
            xsem.at[c % XRING])
        xcp[c].start()

    pcp = {}

    def pstart(layer):
        pcp[layer] = pltpu.make_async_copy(
            proj_hbm.at[layer], pbuf.at[layer % 2], psem.at[layer % 2])
        pcp[layer].start()

    for b in range(ERING):
        estart(b)
    pstart(0)
    for c in range(XRING):
        xstart(c)
    pstart(1)

    pcp[0].wait()
    P = pbuf[0]
    for c in range(NXC):
        xcp[c].wait()
        xg = xbuf[c % XRING][...].reshape(NG, H * D)
        k = _dot(xg, P) * _INV_SQRT_H
        if c + XRING < NXC:
            xstart(c + XRING)
        kbuf[pl.ds(0, NG), :] = k
        b0 = c * (NG // CUR0)
        for j in range(NG // CUR0):
            enc_ref[b0 + j, :] = kbuf[(j + 1) * CUR0 - 1, :]
        xs_ref[pl.ds(c * NG, NG), :] = _dot(k, mem_ref[0])

    def mid_layer(layer, n_in):
        n = n_in // H
        cur = n // B
        xg = xs_ref[pl.ds(0, n_in), :].reshape(n, H * D)
        k = _dot(xg, pbuf[layer % 2]) * _INV_SQRT_H
        if layer + 1 < L:
            pstart(layer + 1)
        kbuf[pl.ds(0, n), :] = k
        for j in range(B):
            enc_ref[j, :] += kbuf[(j + 1) * cur - 1, :]
        xs_ref[pl.ds(0, n), :] = _dot(k, mem_ref[layer])

    pcp[1].wait()
    mid_layer(1, 2048)
    pcp[2].wait()
    mid_layer(2, 512)

    pcp[3].wait()
    xg = xs_ref[pl.ds(0, 128), :].reshape(B, H * D)
    k = _dot(xg, pbuf[1]) * _INV_SQRT_H
    retr = _dot(enc_ref[...] + k, mem_ref[3])

    bval = jnp.full((B,), -jnp.inf, jnp.float32)
    bidx = jnp.zeros((B,), jnp.int32)
    for b in range(NBLK):
        ecp[b].wait()
        scores = _dot_t(retr, ebuf[b % ERING][...])
        if b + ERING < NBLK:
            estart(b + ERING)
        bm = jnp.max(scores, axis=1)
        io = lax.broadcasted_iota(jnp.int32, (B, BV), 1)
        ba = jnp.min(jnp.where(scores == bm[:, None], io, V), axis=1) + b * BV
        better = bm > bval
        bval = jnp.where(better, bm, bval)
        bidx = jnp.where(better, ba, bidx)
    val_out[...] = bval
    idx_out[...] = bidx


def _tc_fused(x, proj2, mem, emb, interpret=False):
    return pl.pallas_call(
        _fused_body,
        in_specs=[
            pl.BlockSpec(memory_space=pltpu.MemorySpace.HBM),
            pl.BlockSpec(memory_space=pltpu.MemorySpace.HBM),
            pl.BlockSpec(memory_space=pltpu.MemorySpace.VMEM),
            pl.BlockSpec(memory_space=pltpu.MemorySpace.HBM),
        ],
        out_specs=[
            pl.BlockSpec(memory_space=pltpu.MemorySpace.VMEM),
            pl.BlockSpec(memory_space=pltpu.MemorySpace.VMEM),
        ],
        out_shape=[
            jax.ShapeDtypeStruct((B,), jnp.float32),
            jax.ShapeDtypeStruct((B,), jnp.int32),
        ],
        scratch_shapes=[
            pltpu.VMEM((XRING, TOK, D), jnp.float32),
            pltpu.VMEM((2, H * D, D), jnp.float32),
            pltpu.VMEM((ERING, BV, D), jnp.float32),
            pltpu.VMEM((2048, D), jnp.float32),
            pltpu.VMEM((512, D), jnp.float32),
            pltpu.VMEM((B, D), jnp.float32),
            pltpu.SemaphoreType.DMA((XRING,)),
            pltpu.SemaphoreType.DMA((2,)),
            pltpu.SemaphoreType.DMA((ERING,)),
        ],
        interpret=interpret,
    )(x, proj2, mem, emb)


def kernel(input_sequence_ids, emb, proj, mem):
    ids = input_sequence_ids.reshape(NW * NCH, GCHUNK).astype(jnp.int32)
    x = _sc_gather()(emb, ids)
    score_max, tok = _tc_fused(x, proj.reshape(L, H * D, D), mem, emb)
    out_vec = _sc_gather_small()(emb, tok)
    return (out_vec, score_max)
